# initial kernel scaffold (unmeasured)
import jax
import jax.numpy as jnp
from jax import lax
from jax.experimental import pallas as pl
from jax.experimental.pallas import tpu as pltpu

X_SIZE = 2
N_CHUNK_PER_B = 2


def kernel(O, Wo):
    B, S, Hs, D = O.shape
    K = Hs * D
    N = Wo.shape[1]
    s_half = S // X_SIZE
    cs = s_half // N_CHUNK_PER_B
    nc = B * N_CHUNK_PER_B

    O3 = O.reshape(B, S, K)

    def body(o_ref, wo_ref, out_ref, send_buf, send_sems, recv_sems):
        my_x = lax.axis_index("x")
        my_y = lax.axis_index("y")
        my_z = lax.axis_index("z")
        partner = (1 - my_x, my_y, my_z)

        my_s0 = my_x * s_half
        their_s0 = (1 - my_x) * s_half

        barrier = pltpu.get_barrier_semaphore()
        pl.semaphore_signal(barrier, inc=1, device_id=partner,
                            device_id_type=pl.DeviceIdType.MESH)
        pl.semaphore_wait(barrier, 1)

        rdmas = []
        for c in range(nc):
            b, sub = divmod(c, N_CHUNK_PER_B)
            so = sub * cs
            slot = c % 2
            if c >= 2:
                rdmas[c - 2].wait_send()
            send_buf[slot, :, :] = jnp.dot(
                o_ref[b, pl.ds(their_s0 + so, cs), :], wo_ref[:, :],
                preferred_element_type=jnp.float32)
            rdma = pltpu.make_async_remote_copy(
                src_ref=send_buf.at[slot],
                dst_ref=out_ref.at[b, pl.ds(so, cs), :],
                send_sem=send_sems.at[slot],
                recv_sem=recv_sems.at[c],
                device_id=partner,
                device_id_type=pl.DeviceIdType.MESH)
            rdma.start()
            rdmas.append(rdma)

        for c in range(nc):
            b, sub = divmod(c, N_CHUNK_PER_B)
            so = sub * cs
            local = jnp.dot(
                o_ref[b, pl.ds(my_s0 + so, cs), :], wo_ref[:, :],
                preferred_element_type=jnp.float32)
            rdmas[c].wait_recv()
            out_ref[b, pl.ds(so, cs), :] = out_ref[b, pl.ds(so, cs), :] + local

        rdmas[nc - 2].wait_send()
        rdmas[nc - 1].wait_send()

    return pl.pallas_call(
        body,
        out_shape=jax.ShapeDtypeStruct((B, s_half, N), jnp.float32),
        in_specs=[pl.BlockSpec(memory_space=pltpu.VMEM),
                  pl.BlockSpec(memory_space=pltpu.VMEM)],
        out_specs=pl.BlockSpec(memory_space=pltpu.VMEM),
        scratch_shapes=[
            pltpu.VMEM((2, cs, N), jnp.float32),
            pltpu.SemaphoreType.DMA((2,)),
            pltpu.SemaphoreType.DMA((nc,)),
        ],
        compiler_params=pltpu.CompilerParams(collective_id=0),
    )(O3, Wo)


# baseline (device time: 469073 ns/iter reference)
import jax
import jax.numpy as jnp
from jax import lax
from jax.experimental import pallas as pl
from jax.experimental.pallas import tpu as pltpu

X_SIZE = 2
CS = 128


def kernel(O, Wo):
    B, S, Hs, D = O.shape
    K = Hs * D
    N = Wo.shape[1]
    s_half = S // X_SIZE
    per_b = s_half // CS
    nc = B * per_b

    O3 = O.reshape(B, S, K)

    def body(o_ref, wo_ref, out_ref, a_buf, send_buf, acc_buf, rtmp,
             a_sems, send_sems, recv_sems, rtmp_sems, wb_sems):
        my_x = lax.axis_index("x")
        my_y = lax.axis_index("y")
        my_z = lax.axis_index("z")
        partner = (1 - my_x, my_y, my_z)

        my_s0 = my_x * s_half
        their_s0 = (1 - my_x) * s_half

        def chunk_bso(c):
            return c // per_b, (c % per_b) * CS

        def out_chunk(c):
            b, so = chunk_bso(c)
            return out_ref.at[b, pl.ds(so, CS), :]

        def a_copy(c, slot, s0):
            b, so = chunk_bso(c)
            return pltpu.make_async_copy(
                o_ref.at[b, pl.ds(s0 + so, CS), :],
                a_buf.at[slot],
                a_sems.at[slot])

        def send_rdma(c, slot):
            return pltpu.make_async_remote_copy(
                src_ref=send_buf.at[slot],
                dst_ref=out_chunk(c),
                send_sem=send_sems.at[slot],
                recv_sem=recv_sems.at[c],
                device_id=partner,
                device_id_type=pl.DeviceIdType.MESH)

        def wb_copy(c, slot):
            return pltpu.make_async_copy(
                acc_buf.at[slot], out_chunk(c), wb_sems.at[slot])

        barrier = pltpu.get_barrier_semaphore()
        pl.semaphore_signal(barrier, inc=1, device_id=partner,
                            device_id_type=pl.DeviceIdType.MESH)
        pl.semaphore_wait(barrier, 1)

        a_copy(0, 0, their_s0).start()
        a_copy(1, 1, their_s0).start()

        def remote_half(p, slot, s0):
            c = 2 * p + slot
            a_copy(c, slot, s0).wait()

            @pl.when(c >= 2)
            def _():
                send_rdma(c - 2, slot).wait_send()

            send_buf[slot, :, :] = jnp.dot(
                a_buf[slot], wo_ref[:, :],
                preferred_element_type=jnp.float32)

            @pl.when(c + 2 < nc)
            def _():
                a_copy(c + 2, slot, s0).start()

            send_rdma(c, slot).start()

        def remote_pair(p, carry):
            remote_half(p, 0, their_s0)
            remote_half(p, 1, their_s0)
            return carry

        lax.fori_loop(0, nc // 2, remote_pair, 0)

        a_copy(0, 0, my_s0).start()
        a_copy(1, 1, my_s0).start()

        def local_half(p, slot):
            c = 2 * p + slot
            a_copy(c, slot, my_s0).wait()

            @pl.when(c >= 2)
            def _():
                wb_copy(c - 2, slot).wait()

            acc_buf[slot, :, :] = jnp.dot(
                a_buf[slot], wo_ref[:, :],
                preferred_element_type=jnp.float32)

            @pl.when(c + 2 < nc)
            def _():
                a_copy(c + 2, slot, my_s0).start()

            send_rdma(c, slot).wait_recv()
            rcp = pltpu.make_async_copy(
                out_chunk(c), rtmp.at[slot], rtmp_sems.at[slot])
            rcp.start()
            rcp.wait()
            acc_buf[slot, :, :] = acc_buf[slot] + rtmp[slot]
            wb_copy(c, slot).start()

        def local_pair(p, carry):
            local_half(p, 0)
            local_half(p, 1)
            return carry

        lax.fori_loop(0, nc // 2, local_pair, 0)

        send_rdma(nc - 2, 0).wait_send()
        send_rdma(nc - 1, 1).wait_send()
        wb_copy(nc - 2, 0).wait()
        wb_copy(nc - 1, 1).wait()

    return pl.pallas_call(
        body,
        out_shape=jax.ShapeDtypeStruct((B, s_half, N), jnp.float32),
        in_specs=[pl.BlockSpec(memory_space=pl.ANY),
                  pl.BlockSpec(memory_space=pltpu.VMEM)],
        out_specs=pl.BlockSpec(memory_space=pl.ANY),
        scratch_shapes=[
            pltpu.VMEM((2, CS, K), jnp.float32),
            pltpu.VMEM((2, CS, N), jnp.float32),
            pltpu.VMEM((2, CS, N), jnp.float32),
            pltpu.VMEM((2, CS, N), jnp.float32),
            pltpu.SemaphoreType.DMA((2,)),
            pltpu.SemaphoreType.DMA((2,)),
            pltpu.SemaphoreType.DMA((nc,)),
            pltpu.SemaphoreType.DMA((2,)),
            pltpu.SemaphoreType.DMA((2,)),
        ],
        compiler_params=pltpu.CompilerParams(
            collective_id=0,
            vmem_limit_bytes=60 * 1024 * 1024),
    )(O3, Wo)


# device time: 440023 ns/iter; 1.0660x vs baseline; 1.0660x over previous
import jax
import jax.numpy as jnp
from jax import lax
from jax.experimental import pallas as pl
from jax.experimental.pallas import tpu as pltpu

X_SIZE = 2
CS = 128


def kernel(O, Wo):
    B, S, Hs, D = O.shape
    K = Hs * D
    N = Wo.shape[1]
    s_half = S // X_SIZE
    per_b = s_half // CS
    nc = B * per_b

    O3 = O.reshape(B, S, K)

    def body(o_ref, wo_ref, out_ref, ar_buf, al_buf, send_buf, acc_buf, rtmp,
             ar_sems, al_sems, send_sems, recv_sems, rtmp_sems, wb_sems):
        my_x = lax.axis_index("x")
        my_y = lax.axis_index("y")
        my_z = lax.axis_index("z")
        partner = (1 - my_x, my_y, my_z)

        my_s0 = my_x * s_half
        their_s0 = (1 - my_x) * s_half

        def chunk_bso(c):
            return c // per_b, (c % per_b) * CS

        def out_chunk(c):
            b, so = chunk_bso(c)
            return out_ref.at[b, pl.ds(so, CS), :]

        def a_copy(c, buf, sems, slot, s0):
            b, so = chunk_bso(c)
            return pltpu.make_async_copy(
                o_ref.at[b, pl.ds(s0 + so, CS), :],
                buf.at[slot], sems.at[slot])

        def send_rdma(c, slot):
            return pltpu.make_async_remote_copy(
                src_ref=send_buf.at[slot],
                dst_ref=out_chunk(c),
                send_sem=send_sems.at[slot],
                recv_sem=recv_sems.at[c],
                device_id=partner,
                device_id_type=pl.DeviceIdType.MESH)

        def wb_copy(c, slot):
            return pltpu.make_async_copy(
                acc_buf.at[slot], out_chunk(c), wb_sems.at[slot])

        def add_received(c, slot):
            send_rdma(c, slot).wait_recv()
            rcp = pltpu.make_async_copy(
                out_chunk(c), rtmp.at[slot], rtmp_sems.at[slot])
            rcp.start()
            rcp.wait()
            acc_buf[slot, :, :] = acc_buf[slot] + rtmp[slot]
            wb_copy(c, slot).start()

        barrier = pltpu.get_barrier_semaphore()
        pl.semaphore_signal(barrier, inc=1, device_id=partner,
                            device_id_type=pl.DeviceIdType.MESH)
        pl.semaphore_wait(barrier, 1)

        a_copy(0, ar_buf, ar_sems, 0, their_s0).start()
        a_copy(1, ar_buf, ar_sems, 1, their_s0).start()
        a_copy(0, al_buf, al_sems, 0, my_s0).start()
        a_copy(1, al_buf, al_sems, 1, my_s0).start()

        def step(p, slot):
            c = 2 * p + slot

            a_copy(c, ar_buf, ar_sems, slot, their_s0).wait()

            @pl.when(c >= 2)
            def _():
                send_rdma(c - 2, slot).wait_send()

            send_buf[slot, :, :] = jnp.dot(
                ar_buf[slot], wo_ref[:, :],
                preferred_element_type=jnp.float32)

            @pl.when(c + 2 < nc)
            def _():
                a_copy(c + 2, ar_buf, ar_sems, slot, their_s0).start()

            send_rdma(c, slot).start()

            a_copy(c, al_buf, al_sems, slot, my_s0).wait()

            @pl.when(c >= 2)
            def _():
                wb_copy(c - 2, slot).wait()

            acc_buf[slot, :, :] = jnp.dot(
                al_buf[slot], wo_ref[:, :],
                preferred_element_type=jnp.float32)

            @pl.when(c + 2 < nc)
            def _():
                a_copy(c + 2, al_buf, al_sems, slot, my_s0).start()

            @pl.when(c >= 1)
            def _():
                add_received(c - 1, 1 - slot)

        def pair(p, carry):
            step(p, 0)
            step(p, 1)
            return carry

        lax.fori_loop(0, nc // 2, pair, 0)

        add_received(nc - 1, (nc - 1) % 2)
        send_rdma(nc - 2, 0).wait_send()
        send_rdma(nc - 1, 1).wait_send()
        wb_copy(nc - 2, 0).wait()
        wb_copy(nc - 1, 1).wait()

    return pl.pallas_call(
        body,
        out_shape=jax.ShapeDtypeStruct((B, s_half, N), jnp.float32),
        in_specs=[pl.BlockSpec(memory_space=pl.ANY),
                  pl.BlockSpec(memory_space=pltpu.VMEM)],
        out_specs=pl.BlockSpec(memory_space=pl.ANY),
        scratch_shapes=[
            pltpu.VMEM((2, CS, K), jnp.float32),
            pltpu.VMEM((2, CS, K), jnp.float32),
            pltpu.VMEM((2, CS, N), jnp.float32),
            pltpu.VMEM((2, CS, N), jnp.float32),
            pltpu.VMEM((2, CS, N), jnp.float32),
            pltpu.SemaphoreType.DMA((2,)),
            pltpu.SemaphoreType.DMA((2,)),
            pltpu.SemaphoreType.DMA((2,)),
            pltpu.SemaphoreType.DMA((nc,)),
            pltpu.SemaphoreType.DMA((2,)),
            pltpu.SemaphoreType.DMA((2,)),
        ],
        compiler_params=pltpu.CompilerParams(
            collective_id=0,
            vmem_limit_bytes=60 * 1024 * 1024),
    )(O3, Wo)
